# baseline (speedup 1.0000x reference)
import numpy as np
import jax
import jax.numpy as jnp
from jax.experimental import pallas as pl
from jax.experimental.pallas import tpu as pltpu

_NO_CLASS = 21
_B, _P, _N = 64, 8732, 16
_G = 8
_HI = jax.lax.Precision.HIGHEST


def _make_prior_cxcy():
    fmap_dims = [38, 19, 10, 5, 3, 1]
    obj_scales = [0.1, 0.2, 0.375, 0.55, 0.725, 0.9]
    aspect_ratios = [[1., 2., 0.5], [1., 2., 3., 0.5, .333], [1., 2., 3., 0.5, .333],
                     [1., 2., 3., 0.5, .333], [1., 2., 0.5], [1., 2., 0.5]]
    priors = []
    for k, dim in enumerate(fmap_dims):
        s = obj_scales[k]
        for i in range(dim):
            for j in range(dim):
                cx = (j + 0.5) / dim
                cy = (i + 0.5) / dim
                for ratio in aspect_ratios[k]:
                    priors.append([cx, cy, s * np.sqrt(ratio), s / np.sqrt(ratio)])
                    if ratio == 1.:
                        if k + 1 < len(obj_scales):
                            add_s = np.sqrt(s * obj_scales[k + 1])
                        else:
                            add_s = 1.
                        priors.append([cx, cy, add_s, add_s])
    return np.clip(np.array(priors, dtype=np.float32), 0., 1.)


_PRIOR_CXCY = _make_prior_cxcy()
_PRIOR_XY = np.concatenate(
    [_PRIOR_CXCY[:, :2] - _PRIOR_CXCY[:, 2:] / 2.,
     _PRIOR_CXCY[:, :2] + _PRIOR_CXCY[:, 2:] / 2.], axis=1).astype(np.float32)
_PXY_T = np.ascontiguousarray(_PRIOR_XY.T)
_PCC_T = np.ascontiguousarray(_PRIOR_CXCY.T)


def _eye(n):
    i = jax.lax.broadcasted_iota(jnp.int32, (n, n), 0)
    j = jax.lax.broadcasted_iota(jnp.int32, (n, n), 1)
    return jnp.where(i == j, 1., 0.).astype(jnp.float32)


def _tr_body(ps_ref, pb_ref, pst_ref, pbt_ref):
    dn = (((1,), (1,)), ((), ()))
    pst_ref[0] = jax.lax.dot_general(
        _eye(_NO_CLASS), ps_ref[0], dn, precision=_HI)
    pbt_ref[0] = jax.lax.dot_general(
        _eye(4), pb_ref[0], dn, precision=_HI)


def _match_body(boxes_ref, labels_ref, pb_ref, ps_ref, pxy_ref, pcc_ref,
                cn_ref, npos_ref, cpos_ref, sl1_ref):
    f32 = jnp.float32
    bx = boxes_ref[...]
    x1, y1, x2, y2 = (bx[:, :, 0:1], bx[:, :, 1:2],
                      bx[:, :, 2:3], bx[:, :, 3:4])
    px1 = pxy_ref[0:1, :][None]
    py1 = pxy_ref[1:2, :][None]
    px2 = pxy_ref[2:3, :][None]
    py2 = pxy_ref[3:4, :][None]

    inter = (jnp.maximum(jnp.minimum(x2, px2) - jnp.maximum(x1, px1), 0.) *
             jnp.maximum(jnp.minimum(y2, py2) - jnp.maximum(y1, py1), 0.))
    area1 = (x2 - x1) * (y2 - y1)
    area2 = (px2 - px1) * (py2 - py1)
    ov = inter / (area1 + area2 - inter)

    ji = jax.lax.broadcasted_iota(jnp.int32, (_G, _N, _P), 1)
    pi = jax.lax.broadcasted_iota(jnp.int32, (_G, _N, _P), 2)
    big = jnp.int32(2 ** 30)

    ov_max = jnp.max(ov, axis=1, keepdims=True)
    obj_fp = jnp.min(jnp.where(ov == ov_max, ji, big), axis=1, keepdims=True)
    ov_omax = jnp.max(ov, axis=2, keepdims=True)
    prior_fo = jnp.min(jnp.where(ov == ov_omax, pi, big), axis=2, keepdims=True)

    j_sel = jnp.max(jnp.where(prior_fo == pi, ji, -1), axis=1, keepdims=True)
    forced = j_sel >= 0
    obj_sel = jnp.where(forced, j_sel, obj_fp)
    ov_sel = jnp.where(forced, f32(1.0), ov_max)

    hit = jnp.where(obj_sel == ji, 1., 0.).astype(f32)

    lab_p = jax.lax.dot_general(labels_ref[...], hit,
                                (((2,), (1,)), ((0,), (0,))),
                                precision=_HI)
    coords = jax.lax.dot_general(bx, hit,
                                 (((1,), (1,)), ((0,), (0,))),
                                 precision=_HI)

    true_cls = jnp.where(ov_sel < 0.5, 0, lab_p.astype(jnp.int32))
    pos = true_cls != 0
    posf = pos.astype(f32)

    mx1 = coords[:, 0:1, :]
    my1 = coords[:, 1:2, :]
    mx2 = coords[:, 2:3, :]
    my2 = coords[:, 3:4, :]
    cx = (mx1 + mx2) / 2.
    cy = (my1 + my2) / 2.
    w = mx2 - mx1
    h = my2 - my1
    pcx = pcc_ref[0:1, :][None]
    pcy = pcc_ref[1:2, :][None]
    pw = pcc_ref[2:3, :][None]
    ph = pcc_ref[3:4, :][None]
    gcx = (cx - pcx) / (pw / 10.)
    gcy = (cy - pcy) / (ph / 10.)
    gw = jnp.log(w / pw) * 5.
    gh = jnp.log(h / ph) * 5.

    pb = pb_ref[...]
    s_acc = jnp.zeros((_G, 1, _P), f32)
    for c, tl in enumerate((gcx, gcy, gw, gh)):
        d = pb[:, c:c + 1, :] - tl
        ad = jnp.abs(d)
        s_acc = s_acc + jnp.where(ad < 1., 0.5 * d * d, ad - 0.5)
    sl1_sum = jnp.sum(s_acc * posf, axis=2, keepdims=True)

    s = ps_ref[...]
    m = jnp.max(s, axis=1, keepdims=True)
    sh = s - m
    logse = jnp.log(jnp.sum(jnp.exp(sh), axis=1, keepdims=True))
    ci = jax.lax.broadcasted_iota(jnp.int32, (_G, _NO_CLASS, _P), 1)
    sh_at = jnp.sum(jnp.where(true_cls == ci, sh, 0.), axis=1, keepdims=True)
    conf = logse - sh_at

    cn_ref[...] = jnp.where(pos, 0., conf)
    npos_ref[...] = jnp.broadcast_to(
        jnp.sum(posf, axis=2, keepdims=True), (_G, 1, 128))
    cpos_ref[...] = jnp.broadcast_to(
        jnp.sum(conf * posf, axis=2, keepdims=True), (_G, 1, 128))
    sl1_ref[...] = jnp.broadcast_to(sl1_sum, (_G, 1, 128))


def _topk_body(cn_ref, npos_ref, cpos_ref, sl1_ref, out_ref):
    x = cn_ref[...]
    npos = npos_ref[:, 0:1]
    k = 3. * npos
    xb = jax.lax.bitcast_convert_type(x, jnp.int32)

    lo0 = jnp.zeros((_B, 1), jnp.int32)
    hi0 = jnp.full((_B, 1), jnp.int32(0x7F800000))

    def step(_, carry):
        lo, hi = carry
        mid = lo + jax.lax.shift_right_logical(hi - lo, 1)
        cnt = jnp.sum(jnp.where(xb > mid, 1., 0.), axis=1, keepdims=True)
        pred = cnt >= k
        return jnp.where(pred, mid + 1, lo), jnp.where(pred, hi, mid)

    lo, _ = jax.lax.fori_loop(0, 31, step, (lo0, hi0))
    t = jax.lax.bitcast_convert_type(lo, jnp.float32)

    gt = x > t
    cnt_gt = jnp.sum(jnp.where(gt, 1., 0.), axis=1, keepdims=True)
    sum_gt = jnp.sum(jnp.where(gt, x, 0.), axis=1, keepdims=True)
    hard = jnp.where(k > 0., sum_gt + (k - cnt_gt) * t, 0.)

    npos_tot = jnp.sum(npos)
    conf_loss = (jnp.sum(cpos_ref[:, 0:1]) + jnp.sum(hard)) / npos_tot
    loc_loss = jnp.sum(sl1_ref[:, 0:1]) / (4. * npos_tot)
    out_ref[...] = jnp.full((1, 128), conf_loss + loc_loss, jnp.float32)


def kernel(pre_box, pre_score, boxes, labels):
    labels_f = labels.astype(jnp.float32).reshape(_B, 1, _N)
    pxy = jnp.asarray(_PXY_T)
    pcc = jnp.asarray(_PCC_T)

    ps_t, pb_t = pl.pallas_call(
        _tr_body,
        grid=(_B,),
        in_specs=[
            pl.BlockSpec((1, _P, _NO_CLASS), lambda i: (i, 0, 0)),
            pl.BlockSpec((1, _P, 4), lambda i: (i, 0, 0)),
        ],
        out_specs=[
            pl.BlockSpec((1, _NO_CLASS, _P), lambda i: (i, 0, 0)),
            pl.BlockSpec((1, 4, _P), lambda i: (i, 0, 0)),
        ],
        out_shape=[
            jax.ShapeDtypeStruct((_B, _NO_CLASS, _P), jnp.float32),
            jax.ShapeDtypeStruct((_B, 4, _P), jnp.float32),
        ],
        compiler_params=pltpu.CompilerParams(
            dimension_semantics=("parallel",)),
    )(pre_score, pre_box)

    cn, npos, cpos, sl1 = pl.pallas_call(
        _match_body,
        grid=(_B // _G,),
        in_specs=[
            pl.BlockSpec((_G, _N, 4), lambda i: (i, 0, 0)),
            pl.BlockSpec((_G, 1, _N), lambda i: (i, 0, 0)),
            pl.BlockSpec((_G, 4, _P), lambda i: (i, 0, 0)),
            pl.BlockSpec((_G, _NO_CLASS, _P), lambda i: (i, 0, 0)),
            pl.BlockSpec((4, _P), lambda i: (0, 0)),
            pl.BlockSpec((4, _P), lambda i: (0, 0)),
        ],
        out_specs=[
            pl.BlockSpec((_G, 1, _P), lambda i: (i, 0, 0)),
            pl.BlockSpec((_G, 1, 128), lambda i: (i, 0, 0)),
            pl.BlockSpec((_G, 1, 128), lambda i: (i, 0, 0)),
            pl.BlockSpec((_G, 1, 128), lambda i: (i, 0, 0)),
        ],
        out_shape=[
            jax.ShapeDtypeStruct((_B, 1, _P), jnp.float32),
            jax.ShapeDtypeStruct((_B, 1, 128), jnp.float32),
            jax.ShapeDtypeStruct((_B, 1, 128), jnp.float32),
            jax.ShapeDtypeStruct((_B, 1, 128), jnp.float32),
        ],
        compiler_params=pltpu.CompilerParams(
            dimension_semantics=("parallel",)),
    )(boxes, labels_f, pb_t, ps_t, pxy, pcc)

    out = pl.pallas_call(
        _topk_body,
        out_shape=jax.ShapeDtypeStruct((1, 128), jnp.float32),
    )(cn.reshape(_B, _P), npos.reshape(_B, 128),
      cpos.reshape(_B, 128), sl1.reshape(_B, 128))
    return out[0, 0]


# speedup vs baseline: 3.8664x; 3.8664x over previous
import numpy as np
import jax
import jax.numpy as jnp
from jax.experimental import pallas as pl
from jax.experimental.pallas import tpu as pltpu

_NO_CLASS = 21
_B, _P, _N = 64, 8732, 16
_G = 8
_HI = jax.lax.Precision.HIGHEST


def _make_prior_cxcy():
    fmap_dims = [38, 19, 10, 5, 3, 1]
    obj_scales = [0.1, 0.2, 0.375, 0.55, 0.725, 0.9]
    aspect_ratios = [[1., 2., 0.5], [1., 2., 3., 0.5, .333], [1., 2., 3., 0.5, .333],
                     [1., 2., 3., 0.5, .333], [1., 2., 0.5], [1., 2., 0.5]]
    priors = []
    for k, dim in enumerate(fmap_dims):
        s = obj_scales[k]
        for i in range(dim):
            for j in range(dim):
                cx = (j + 0.5) / dim
                cy = (i + 0.5) / dim
                for ratio in aspect_ratios[k]:
                    priors.append([cx, cy, s * np.sqrt(ratio), s / np.sqrt(ratio)])
                    if ratio == 1.:
                        if k + 1 < len(obj_scales):
                            add_s = np.sqrt(s * obj_scales[k + 1])
                        else:
                            add_s = 1.
                        priors.append([cx, cy, add_s, add_s])
    return np.clip(np.array(priors, dtype=np.float32), 0., 1.)


_PRIOR_CXCY = _make_prior_cxcy()
_PRIOR_XY = np.concatenate(
    [_PRIOR_CXCY[:, :2] - _PRIOR_CXCY[:, 2:] / 2.,
     _PRIOR_CXCY[:, :2] + _PRIOR_CXCY[:, 2:] / 2.], axis=1).astype(np.float32)
_PXY_T = np.ascontiguousarray(_PRIOR_XY.T)
_PCC_T = np.ascontiguousarray(_PRIOR_CXCY.T)


def _eye(n):
    i = jax.lax.broadcasted_iota(jnp.int32, (n, n), 0)
    j = jax.lax.broadcasted_iota(jnp.int32, (n, n), 1)
    return jnp.where(i == j, 1., 0.).astype(jnp.float32)


def _tr_body(ps_ref, pb_ref, pst_ref, pbt_ref):
    dn = (((1,), (1,)), ((), ()))
    pst_ref[0] = jax.lax.dot_general(
        _eye(_NO_CLASS), ps_ref[0], dn, precision=_HI)
    pbt_ref[0] = jax.lax.dot_general(
        _eye(4), pb_ref[0], dn, precision=_HI)


def _match_body(boxes_ref, labels_ref, pb_ref, ps_ref, pxy_ref, pcc_ref,
                cn_ref, npos_ref, cpos_ref, sl1_ref):
    f32 = jnp.float32
    bx = boxes_ref[...]
    x1, y1, x2, y2 = (bx[:, :, 0:1], bx[:, :, 1:2],
                      bx[:, :, 2:3], bx[:, :, 3:4])
    px1 = pxy_ref[0:1, :][None]
    py1 = pxy_ref[1:2, :][None]
    px2 = pxy_ref[2:3, :][None]
    py2 = pxy_ref[3:4, :][None]

    inter = (jnp.maximum(jnp.minimum(x2, px2) - jnp.maximum(x1, px1), 0.) *
             jnp.maximum(jnp.minimum(y2, py2) - jnp.maximum(y1, py1), 0.))
    area1 = (x2 - x1) * (y2 - y1)
    area2 = (px2 - px1) * (py2 - py1)
    ov = inter / (area1 + area2 - inter)

    ji = jax.lax.broadcasted_iota(jnp.int32, (_G, _N, _P), 1)
    pi = jax.lax.broadcasted_iota(jnp.int32, (_G, _N, _P), 2)
    big = jnp.int32(2 ** 30)

    ov_max = jnp.max(ov, axis=1, keepdims=True)
    obj_fp = jnp.min(jnp.where(ov == ov_max, ji, big), axis=1, keepdims=True)
    ov_omax = jnp.max(ov, axis=2, keepdims=True)
    prior_fo = jnp.min(jnp.where(ov == ov_omax, pi, big), axis=2, keepdims=True)

    j_sel = jnp.max(jnp.where(prior_fo == pi, ji, -1), axis=1, keepdims=True)
    forced = j_sel >= 0
    obj_sel = jnp.where(forced, j_sel, obj_fp)
    ov_sel = jnp.where(forced, f32(1.0), ov_max)

    hit = jnp.where(obj_sel == ji, 1., 0.).astype(f32)

    lab_p = jax.lax.dot_general(labels_ref[...], hit,
                                (((2,), (1,)), ((0,), (0,))),
                                precision=_HI)
    coords = jax.lax.dot_general(bx, hit,
                                 (((1,), (1,)), ((0,), (0,))),
                                 precision=_HI)

    true_cls = jnp.where(ov_sel < 0.5, 0, lab_p.astype(jnp.int32))
    pos = true_cls != 0
    posf = pos.astype(f32)

    mx1 = coords[:, 0:1, :]
    my1 = coords[:, 1:2, :]
    mx2 = coords[:, 2:3, :]
    my2 = coords[:, 3:4, :]
    cx = (mx1 + mx2) / 2.
    cy = (my1 + my2) / 2.
    w = mx2 - mx1
    h = my2 - my1
    pcx = pcc_ref[0:1, :][None]
    pcy = pcc_ref[1:2, :][None]
    pw = pcc_ref[2:3, :][None]
    ph = pcc_ref[3:4, :][None]
    gcx = (cx - pcx) / (pw / 10.)
    gcy = (cy - pcy) / (ph / 10.)
    gw = jnp.log(w / pw) * 5.
    gh = jnp.log(h / ph) * 5.

    pb = pb_ref[...]
    s_acc = jnp.zeros((_G, 1, _P), f32)
    for c, tl in enumerate((gcx, gcy, gw, gh)):
        d = pb[:, c:c + 1, :] - tl
        ad = jnp.abs(d)
        s_acc = s_acc + jnp.where(ad < 1., 0.5 * d * d, ad - 0.5)
    sl1_sum = jnp.sum(s_acc * posf, axis=2, keepdims=True)

    s = ps_ref[...]
    m = jnp.max(s, axis=1, keepdims=True)
    sh = s - m
    logse = jnp.log(jnp.sum(jnp.exp(sh), axis=1, keepdims=True))
    ci = jax.lax.broadcasted_iota(jnp.int32, (_G, _NO_CLASS, _P), 1)
    sh_at = jnp.sum(jnp.where(true_cls == ci, sh, 0.), axis=1, keepdims=True)
    conf = logse - sh_at

    cn_ref[...] = jnp.where(pos, 0., conf)
    npos_ref[...] = jnp.broadcast_to(
        jnp.sum(posf, axis=2, keepdims=True), (_G, 1, 128))
    cpos_ref[...] = jnp.broadcast_to(
        jnp.sum(conf * posf, axis=2, keepdims=True), (_G, 1, 128))
    sl1_ref[...] = jnp.broadcast_to(sl1_sum, (_G, 1, 128))


def _topk_body(cn_ref, npos_ref, cpos_ref, sl1_ref, out_ref):
    x = cn_ref[...]
    npos = npos_ref[:, 0:1]
    k = 3. * npos
    xb = jax.lax.bitcast_convert_type(x, jnp.int32)

    lo0 = jnp.zeros((_B, 1), jnp.int32)
    hi0 = jnp.full((_B, 1), jnp.int32(0x7F800000))

    def step(_, carry):
        lo, hi = carry
        mid = lo + jax.lax.shift_right_logical(hi - lo, 1)
        cnt = jnp.sum(jnp.where(xb > mid, 1., 0.), axis=1, keepdims=True)
        pred = cnt >= k
        return jnp.where(pred, mid + 1, lo), jnp.where(pred, hi, mid)

    lo, _ = jax.lax.fori_loop(0, 31, step, (lo0, hi0))
    t = jax.lax.bitcast_convert_type(lo, jnp.float32)

    gt = x > t
    cnt_gt = jnp.sum(jnp.where(gt, 1., 0.), axis=1, keepdims=True)
    sum_gt = jnp.sum(jnp.where(gt, x, 0.), axis=1, keepdims=True)
    hard = jnp.where(k > 0., sum_gt + (k - cnt_gt) * t, 0.)

    npos_tot = jnp.sum(npos)
    conf_loss = (jnp.sum(cpos_ref[:, 0:1]) + jnp.sum(hard)) / npos_tot
    loc_loss = jnp.sum(sl1_ref[:, 0:1]) / (4. * npos_tot)
    out_ref[...] = jnp.full((1, 128), conf_loss + loc_loss, jnp.float32)


def kernel(pre_box, pre_score, boxes, labels):
    labels_f = labels.astype(jnp.float32).reshape(_B, 1, _N)
    pxy = jnp.asarray(_PXY_T)
    pcc = jnp.asarray(_PCC_T)

    ps_t = jnp.transpose(pre_score, (0, 2, 1))
    pb_t = jnp.transpose(pre_box, (0, 2, 1))

    cn, npos, cpos, sl1 = pl.pallas_call(
        _match_body,
        grid=(_B // _G,),
        in_specs=[
            pl.BlockSpec((_G, _N, 4), lambda i: (i, 0, 0)),
            pl.BlockSpec((_G, 1, _N), lambda i: (i, 0, 0)),
            pl.BlockSpec((_G, 4, _P), lambda i: (i, 0, 0)),
            pl.BlockSpec((_G, _NO_CLASS, _P), lambda i: (i, 0, 0)),
            pl.BlockSpec((4, _P), lambda i: (0, 0)),
            pl.BlockSpec((4, _P), lambda i: (0, 0)),
        ],
        out_specs=[
            pl.BlockSpec((_G, 1, _P), lambda i: (i, 0, 0)),
            pl.BlockSpec((_G, 1, 128), lambda i: (i, 0, 0)),
            pl.BlockSpec((_G, 1, 128), lambda i: (i, 0, 0)),
            pl.BlockSpec((_G, 1, 128), lambda i: (i, 0, 0)),
        ],
        out_shape=[
            jax.ShapeDtypeStruct((_B, 1, _P), jnp.float32),
            jax.ShapeDtypeStruct((_B, 1, 128), jnp.float32),
            jax.ShapeDtypeStruct((_B, 1, 128), jnp.float32),
            jax.ShapeDtypeStruct((_B, 1, 128), jnp.float32),
        ],
        compiler_params=pltpu.CompilerParams(
            dimension_semantics=("parallel",)),
    )(boxes, labels_f, pb_t, ps_t, pxy, pcc)

    out = pl.pallas_call(
        _topk_body,
        out_shape=jax.ShapeDtypeStruct((1, 128), jnp.float32),
    )(cn.reshape(_B, _P), npos.reshape(_B, 128),
      cpos.reshape(_B, 128), sl1.reshape(_B, 128))
    return out[0, 0]


# speedup vs baseline: 4.2089x; 1.0886x over previous
import numpy as np
import jax
import jax.numpy as jnp
from jax.experimental import pallas as pl
from jax.experimental.pallas import tpu as pltpu

_NO_CLASS = 21
_B, _P, _N = 64, 8732, 16
_G = 8


def _make_prior_cxcy():
    fmap_dims = [38, 19, 10, 5, 3, 1]
    obj_scales = [0.1, 0.2, 0.375, 0.55, 0.725, 0.9]
    aspect_ratios = [[1., 2., 0.5], [1., 2., 3., 0.5, .333], [1., 2., 3., 0.5, .333],
                     [1., 2., 3., 0.5, .333], [1., 2., 0.5], [1., 2., 0.5]]
    priors = []
    for k, dim in enumerate(fmap_dims):
        s = obj_scales[k]
        for i in range(dim):
            for j in range(dim):
                cx = (j + 0.5) / dim
                cy = (i + 0.5) / dim
                for ratio in aspect_ratios[k]:
                    priors.append([cx, cy, s * np.sqrt(ratio), s / np.sqrt(ratio)])
                    if ratio == 1.:
                        if k + 1 < len(obj_scales):
                            add_s = np.sqrt(s * obj_scales[k + 1])
                        else:
                            add_s = 1.
                        priors.append([cx, cy, add_s, add_s])
    return np.clip(np.array(priors, dtype=np.float32), 0., 1.)


_PRIOR_CXCY = _make_prior_cxcy()
_PRIOR_XY = np.concatenate(
    [_PRIOR_CXCY[:, :2] - _PRIOR_CXCY[:, 2:] / 2.,
     _PRIOR_CXCY[:, :2] + _PRIOR_CXCY[:, 2:] / 2.], axis=1).astype(np.float32)
_PXY_T = np.ascontiguousarray(_PRIOR_XY.T)
_PCC_T = np.ascontiguousarray(_PRIOR_CXCY.T)


def _eye(n):
    i = jax.lax.broadcasted_iota(jnp.int32, (n, n), 0)
    j = jax.lax.broadcasted_iota(jnp.int32, (n, n), 1)
    return jnp.where(i == j, 1., 0.).astype(jnp.float32)


def _match_body(baug_ref, pb_ref, ps_ref, pxy_ref, pcc_ref,
                cn_ref, npos_ref, cpos_ref, sl1_ref):
    f32 = jnp.float32
    bx = baug_ref[...][:, :, 0:4]
    x1, y1, x2, y2 = (bx[:, :, 0:1], bx[:, :, 1:2],
                      bx[:, :, 2:3], bx[:, :, 3:4])
    px1 = pxy_ref[0:1, :][None]
    py1 = pxy_ref[1:2, :][None]
    px2 = pxy_ref[2:3, :][None]
    py2 = pxy_ref[3:4, :][None]

    inter = (jnp.maximum(jnp.minimum(x2, px2) - jnp.maximum(x1, px1), 0.) *
             jnp.maximum(jnp.minimum(y2, py2) - jnp.maximum(y1, py1), 0.))
    area1 = (x2 - x1) * (y2 - y1)
    area2 = (px2 - px1) * (py2 - py1)
    ov = inter / (area1 + area2 - inter)

    ji = jax.lax.broadcasted_iota(jnp.int32, (_G, _N, _P), 1)
    pi = jax.lax.broadcasted_iota(jnp.int32, (_G, _N, _P), 2)
    big = jnp.int32(2 ** 30)

    ov_max = jnp.max(ov, axis=1, keepdims=True)
    obj_fp = jnp.min(jnp.where(ov == ov_max, ji, big), axis=1, keepdims=True)
    ov_omax = jnp.max(ov, axis=2, keepdims=True)
    prior_fo = jnp.min(jnp.where(ov == ov_omax, pi, big), axis=2, keepdims=True)

    j_sel = jnp.max(jnp.where(prior_fo == pi, ji, -1), axis=1, keepdims=True)
    forced = j_sel >= 0
    obj_sel = jnp.where(forced, j_sel, obj_fp)
    ov_sel = jnp.where(forced, f32(1.0), ov_max)

    hit = jnp.where(obj_sel == ji, 1., 0.).astype(f32)

    aug = jax.lax.dot_general(baug_ref[...], hit,
                              (((1,), (1,)), ((0,), (0,))))
    coords = aug[:, 0:4, :]
    lab_p = aug[:, 4:5, :]

    true_cls = jnp.where(ov_sel < 0.5, 0, lab_p.astype(jnp.int32))
    pos = true_cls != 0
    posf = pos.astype(f32)

    mx1 = coords[:, 0:1, :]
    my1 = coords[:, 1:2, :]
    mx2 = coords[:, 2:3, :]
    my2 = coords[:, 3:4, :]
    cx = (mx1 + mx2) / 2.
    cy = (my1 + my2) / 2.
    w = mx2 - mx1
    h = my2 - my1
    pcx = pcc_ref[0:1, :][None]
    pcy = pcc_ref[1:2, :][None]
    pw = pcc_ref[2:3, :][None]
    ph = pcc_ref[3:4, :][None]
    gcx = (cx - pcx) / (pw / 10.)
    gcy = (cy - pcy) / (ph / 10.)
    gw = jnp.log(w / pw) * 5.
    gh = jnp.log(h / ph) * 5.

    pb = pb_ref[...]
    s_acc = jnp.zeros((_G, 1, _P), f32)
    for c, tl in enumerate((gcx, gcy, gw, gh)):
        d = pb[:, c:c + 1, :] - tl
        ad = jnp.abs(d)
        s_acc = s_acc + jnp.where(ad < 1., 0.5 * d * d, ad - 0.5)
    sl1_sum = jnp.sum(s_acc * posf, axis=2, keepdims=True)

    s = ps_ref[...]
    m = jnp.max(s, axis=1, keepdims=True)
    sh = s - m
    logse = jnp.log(jnp.sum(jnp.exp(sh), axis=1, keepdims=True))
    ci = jax.lax.broadcasted_iota(jnp.int32, (_G, _NO_CLASS, _P), 1)
    sh_at = jnp.sum(jnp.where(true_cls == ci, sh, 0.), axis=1, keepdims=True)
    conf = logse - sh_at

    cn_ref[...] = jnp.where(pos, 0., conf)
    npos_ref[...] = jnp.broadcast_to(
        jnp.sum(posf, axis=2, keepdims=True), (_G, 1, 128))
    cpos_ref[...] = jnp.broadcast_to(
        jnp.sum(conf * posf, axis=2, keepdims=True), (_G, 1, 128))
    sl1_ref[...] = jnp.broadcast_to(sl1_sum, (_G, 1, 128))


def _topk_body(cna_ref, cnb_ref, npa_ref, npb_ref, cpa_ref, cpb_ref,
               sla_ref, slb_ref, out_ref):
    x = jnp.concatenate([cna_ref[...], cnb_ref[...]], axis=0)
    npos = jnp.concatenate([npa_ref[:, 0:1], npb_ref[:, 0:1]], axis=0)
    k = 3. * npos
    xb = jax.lax.bitcast_convert_type(x, jnp.int32)

    lo0 = jnp.zeros((_B, 1), jnp.int32)
    hi0 = jnp.full((_B, 1), jnp.int32(0x7F800000))

    def step(_, carry):
        lo, hi = carry
        mid = lo + jax.lax.shift_right_logical(hi - lo, 1)
        cnt = jnp.sum(jnp.where(xb > mid, 1., 0.), axis=1, keepdims=True)
        pred = cnt >= k
        return jnp.where(pred, mid + 1, lo), jnp.where(pred, hi, mid)

    lo, _ = jax.lax.fori_loop(0, 31, step, (lo0, hi0))
    t = jax.lax.bitcast_convert_type(lo, jnp.float32)

    gt = x > t
    cnt_gt = jnp.sum(jnp.where(gt, 1., 0.), axis=1, keepdims=True)
    sum_gt = jnp.sum(jnp.where(gt, x, 0.), axis=1, keepdims=True)
    hard = jnp.where(k > 0., sum_gt + (k - cnt_gt) * t, 0.)

    npos_tot = jnp.sum(npos)
    cpos_tot = jnp.sum(cpa_ref[:, 0:1]) + jnp.sum(cpb_ref[:, 0:1])
    sl1_tot = jnp.sum(sla_ref[:, 0:1]) + jnp.sum(slb_ref[:, 0:1])
    conf_loss = (cpos_tot + jnp.sum(hard)) / npos_tot
    loc_loss = sl1_tot / (4. * npos_tot)
    out_ref[...] = jnp.full((1, 128), conf_loss + loc_loss, jnp.float32)


def _match_half(baug_h, pb_th, ps_th, pxy, pcc):
    h = _B // 2
    return pl.pallas_call(
        _match_body,
        grid=(h // _G,),
        in_specs=[
            pl.BlockSpec((_G, _N, 5), lambda i: (i, 0, 0)),
            pl.BlockSpec((_G, 4, _P), lambda i: (i, 0, 0)),
            pl.BlockSpec((_G, _NO_CLASS, _P), lambda i: (i, 0, 0)),
            pl.BlockSpec((4, _P), lambda i: (0, 0)),
            pl.BlockSpec((4, _P), lambda i: (0, 0)),
        ],
        out_specs=[
            pl.BlockSpec((_G, 1, _P), lambda i: (i, 0, 0)),
            pl.BlockSpec((_G, 1, 128), lambda i: (i, 0, 0)),
            pl.BlockSpec((_G, 1, 128), lambda i: (i, 0, 0)),
            pl.BlockSpec((_G, 1, 128), lambda i: (i, 0, 0)),
        ],
        out_shape=[
            jax.ShapeDtypeStruct((h, 1, _P), jnp.float32),
            jax.ShapeDtypeStruct((h, 1, 128), jnp.float32),
            jax.ShapeDtypeStruct((h, 1, 128), jnp.float32),
            jax.ShapeDtypeStruct((h, 1, 128), jnp.float32),
        ],
        compiler_params=pltpu.CompilerParams(
            dimension_semantics=("parallel",)),
    )(baug_h, pb_th, ps_th, pxy, pcc)


def kernel(pre_box, pre_score, boxes, labels):
    h = _B // 2
    pxy = jnp.asarray(_PXY_T)
    pcc = jnp.asarray(_PCC_T)
    baug = jnp.concatenate(
        [boxes, labels.astype(jnp.float32)[..., None]], axis=2)

    halves = []
    for i in range(2):
        sl = slice(i * h, (i + 1) * h)
        ps_t = jnp.transpose(pre_score[sl], (0, 2, 1))
        pb_t = jnp.transpose(pre_box[sl], (0, 2, 1))
        halves.append(_match_half(baug[sl], pb_t, ps_t, pxy, pcc))
    (cna, npa, cpa, sla), (cnb, npb, cpb, slb) = halves

    out = pl.pallas_call(
        _topk_body,
        out_shape=jax.ShapeDtypeStruct((1, 128), jnp.float32),
    )(cna.reshape(h, _P), cnb.reshape(h, _P),
      npa.reshape(h, 128), npb.reshape(h, 128),
      cpa.reshape(h, 128), cpb.reshape(h, 128),
      sla.reshape(h, 128), slb.reshape(h, 128))
    return out[0, 0]
